# baseline (speedup 1.0000x reference)
import functools

import jax
import jax.numpy as jnp
from jax import lax
from jax.experimental import pallas as pl
from jax.experimental.pallas import tpu as pltpu
from jax.experimental.pallas import tpu_sc as plsc

L = 16
NC = 2
NS = 16
NW = NC * NS
DIM = 64
CHUNK = 128
NBUF = 8
GROUPS = DIM // L
NTYPES = 3


def _sc_body(total, nums_hbm, types_hbm, ntab_hbm, ttab_hbm, out_hbm,
             idx_v, tidx_v, ttab_v, rows_v, gs, ss):
    c = lax.axis_index("c")
    s = lax.axis_index("s")
    wid = s * NC + c
    per_w = total // NW
    n_chunks = per_w // CHUNK
    base_w = wid * per_w

    pltpu.sync_copy(nums_hbm.at[pl.ds(base_w, per_w)], idx_v)
    pltpu.sync_copy(types_hbm.at[pl.ds(base_w, per_w)],
                    tidx_v.at[pl.ds(0, per_w)])
    pltpu.sync_copy(ttab_hbm, ttab_v)

    def gather(ci, b):
        return pltpu.async_copy(
            ntab_hbm.at[idx_v.at[pl.ds(ci * CHUNK, CHUNK)]],
            rows_v.at[b], gs.at[b])

    def gather_wait(ci, b):
        pltpu.make_async_copy(
            ntab_hbm.at[idx_v.at[pl.ds(ci * CHUNK, CHUNK)]],
            rows_v.at[b], gs.at[b]).wait()

    def add_types(ci, b):
        off = ci * CHUNK

        def row_body(r, carry):
            t = tidx_v[pl.ds(off + r, L)][0]
            for g in range(GROUPS):
                sl = pl.ds(g * L, L)
                rows_v[b, r, sl] = rows_v[b, r, sl] + ttab_v[t, sl]
            return carry

        lax.fori_loop(0, CHUNK, row_body, 0)

    def store(ci, b):
        return pltpu.async_copy(
            rows_v.at[b], out_hbm.at[pl.ds(base_w + ci * CHUNK, CHUNK)],
            ss.at[b])

    def store_wait(ci, b):
        pltpu.make_async_copy(
            rows_v.at[b], out_hbm.at[pl.ds(base_w + ci * CHUNK, CHUNK)],
            ss.at[b]).wait()

    last = n_chunks - 1
    DEPTH = NBUF - 2

    for b in range(DEPTH):
        gather(b, b)
    store(NBUF - 2, NBUF - 2)
    store(NBUF - 1, NBUF - 1)

    def round_body(r, carry):
        for b in range(NBUF):
            ci = r * NBUF + b
            with jax.named_scope("gwait"):
                gather_wait(ci, b)
            with jax.named_scope("add_types"):
                add_types(ci, b)
            with jax.named_scope("sissue"):
                store(ci, b)
            bw = (b - 2) % NBUF
            with jax.named_scope("swait"):
                store_wait((ci - 2) % n_chunks, bw)
            bg = (b + DEPTH) % NBUF
            with jax.named_scope("gissue"):
                gather(jnp.minimum(ci + DEPTH, last), bg)
        return carry

    lax.fori_loop(0, n_chunks // NBUF, round_body, 0)

    for k in range(DEPTH):
        b = (n_chunks + k) % NBUF
        gather_wait(last, b)
    for k in range(2):
        b = (n_chunks - 2 + k) % NBUF
        store_wait(n_chunks - 2 + k, b)


def kernel(task_nums, task_types, task_num_table, task_type_table):
    B, T = task_nums.shape
    total = B * T
    nums = task_nums.reshape(total).astype(jnp.int32)
    types = task_types.reshape(total).astype(jnp.int32)
    per_w = total // NW

    mesh = plsc.VectorSubcoreMesh(core_axis_name="c", subcore_axis_name="s")
    call = pl.kernel(
        functools.partial(_sc_body, total),
        out_type=jax.ShapeDtypeStruct((total, DIM), jnp.float32),
        mesh=mesh,
        scratch_types=[
            pltpu.VMEM((per_w,), jnp.int32),
            pltpu.VMEM((per_w + L,), jnp.int32),
            pltpu.VMEM((NTYPES, DIM), jnp.float32),
            pltpu.VMEM((NBUF, CHUNK, DIM), jnp.float32),
            pltpu.SemaphoreType.DMA((NBUF,)),
            pltpu.SemaphoreType.DMA((NBUF,)),
        ],
        compiler_params=pltpu.CompilerParams(use_tc_tiling_on_sc=False),
    )
    out = call(nums, types, task_num_table, task_type_table)
    return out.reshape(B, T, DIM)


# speedup vs baseline: 1.8839x; 1.8839x over previous
import functools

import jax
import jax.numpy as jnp
from jax import lax
from jax.experimental import pallas as pl
from jax.experimental.pallas import tpu as pltpu
from jax.experimental.pallas import tpu_sc as plsc

L = 16
NC = 2
NS = 16
NW = NC * NS
DIM = 64
CHUNK = 128
NBUF = 8
GROUPS = DIM // L
NTYPES = 3


def _sc_body(total, nums_hbm, types_hbm, ntab_hbm, ttab_hbm, out_hbm,
             idx_v, tidx_v, ttab_v, rows_v, gs, ss):
    c = lax.axis_index("c")
    s = lax.axis_index("s")
    wid = s * NC + c
    per_w = total // NW
    n_chunks = per_w // CHUNK
    base_w = wid * per_w

    pltpu.sync_copy(nums_hbm.at[pl.ds(base_w, per_w)], idx_v)
    pltpu.sync_copy(types_hbm.at[pl.ds(base_w, per_w)],
                    tidx_v.at[pl.ds(0, per_w)])
    pltpu.sync_copy(ttab_hbm, ttab_v)

    def gather(ci, b):
        return pltpu.async_copy(
            ntab_hbm.at[idx_v.at[pl.ds(ci * CHUNK, CHUNK)]],
            rows_v.at[b], gs.at[b])

    def gather_wait(ci, b):
        pltpu.make_async_copy(
            ntab_hbm.at[idx_v.at[pl.ds(ci * CHUNK, CHUNK)]],
            rows_v.at[b], gs.at[b]).wait()

    trow = [[ttab_v[t, pl.ds(g * L, L)] for g in range(GROUPS)]
            for t in range(NTYPES)]

    def add_types(ci, b):
        off = ci * CHUNK

        def blk_body(r16, carry):
            r0 = r16 * L
            t16 = tidx_v[pl.ds(off + r0, L)]
            for j in range(L):
                t = t16[j]
                p0 = t == 0
                p1 = t == 1
                r = r0 + j
                for g in range(GROUPS):
                    sl = pl.ds(g * L, L)
                    add = jnp.where(p0, trow[0][g],
                                    jnp.where(p1, trow[1][g], trow[2][g]))
                    rows_v[b, r, sl] = rows_v[b, r, sl] + add
            return carry

        return blk_body

    def add_types_loop(ci, b):
        lax.fori_loop(0, CHUNK // L, add_types(ci, b), 0)

    def store(ci, b):
        return pltpu.async_copy(
            rows_v.at[b], out_hbm.at[pl.ds(base_w + ci * CHUNK, CHUNK)],
            ss.at[b])

    def store_wait(ci, b):
        pltpu.make_async_copy(
            rows_v.at[b], out_hbm.at[pl.ds(base_w + ci * CHUNK, CHUNK)],
            ss.at[b]).wait()

    last = n_chunks - 1
    DEPTH = NBUF - 2

    for b in range(DEPTH):
        gather(b, b)
    store(NBUF - 2, NBUF - 2)
    store(NBUF - 1, NBUF - 1)

    def round_body(r, carry):
        for b in range(NBUF):
            ci = r * NBUF + b
            with jax.named_scope("gwait"):
                gather_wait(ci, b)
            with jax.named_scope("add_types"):
                add_types_loop(ci, b)
            with jax.named_scope("sissue"):
                store(ci, b)
            bw = (b - 2) % NBUF
            with jax.named_scope("swait"):
                store_wait((ci - 2) % n_chunks, bw)
            bg = (b + DEPTH) % NBUF
            with jax.named_scope("gissue"):
                gather(jnp.minimum(ci + DEPTH, last), bg)
        return carry

    lax.fori_loop(0, n_chunks // NBUF, round_body, 0)

    for k in range(DEPTH):
        b = (n_chunks + k) % NBUF
        gather_wait(last, b)
    for k in range(2):
        b = (n_chunks - 2 + k) % NBUF
        store_wait(n_chunks - 2 + k, b)


def kernel(task_nums, task_types, task_num_table, task_type_table):
    B, T = task_nums.shape
    total = B * T
    nums = task_nums.reshape(total).astype(jnp.int32)
    types = task_types.reshape(total).astype(jnp.int32)
    per_w = total // NW

    mesh = plsc.VectorSubcoreMesh(core_axis_name="c", subcore_axis_name="s")
    call = pl.kernel(
        functools.partial(_sc_body, total),
        out_type=jax.ShapeDtypeStruct((total, DIM), jnp.float32),
        mesh=mesh,
        scratch_types=[
            pltpu.VMEM((per_w,), jnp.int32),
            pltpu.VMEM((per_w + L,), jnp.int32),
            pltpu.VMEM((NTYPES, DIM), jnp.float32),
            pltpu.VMEM((NBUF, CHUNK, DIM), jnp.float32),
            pltpu.SemaphoreType.DMA((NBUF,)),
            pltpu.SemaphoreType.DMA((NBUF,)),
        ],
        compiler_params=pltpu.CompilerParams(use_tc_tiling_on_sc=False),
    )
    out = call(nums, types, task_num_table, task_type_table)
    return out.reshape(B, T, DIM)
